# baseline (speedup 1.0000x reference)
import functools

import numpy as np
import jax
import jax.numpy as jnp
from jax.experimental import pallas as pl

_ROT_A = (13, 15, 26, 6)
_ROT_B = (17, 29, 16, 24)
_KEY0 = np.uint32(0)
_KEY1 = np.uint32(42)
_KEY2 = np.uint32(0x1BD11BDA) ^ _KEY0 ^ _KEY1
_TINY = np.float32(np.finfo(np.float32).tiny)
_SCALE = np.float32(np.float32(1.0) - _TINY)


def _rotl(x, d):
    return (x << np.uint32(d)) | (x >> np.uint32(32 - d))


def _rounds(x0, x1, rots):
    for r in rots:
        x0 = x0 + x1
        x1 = _rotl(x1, r)
        x1 = x0 ^ x1
    return x0, x1


def _threefry_bits(ctr_lo):
    x0 = jnp.zeros_like(ctr_lo) + _KEY0
    x1 = ctr_lo + _KEY1
    x0, x1 = _rounds(x0, x1, _ROT_A)
    x0, x1 = x0 + _KEY1, x1 + (_KEY2 + np.uint32(1))
    x0, x1 = _rounds(x0, x1, _ROT_B)
    x0, x1 = x0 + _KEY2, x1 + (_KEY0 + np.uint32(2))
    x0, x1 = _rounds(x0, x1, _ROT_A)
    x0, x1 = x0 + _KEY0, x1 + (_KEY1 + np.uint32(3))
    x0, x1 = _rounds(x0, x1, _ROT_B)
    x0, x1 = x0 + _KEY1, x1 + (_KEY2 + np.uint32(4))
    x0, x1 = _rounds(x0, x1, _ROT_A)
    x0, x1 = x0 + _KEY2, x1 + (_KEY0 + np.uint32(5))
    return x0 ^ x1


def _sampler_kernel(x_ref, o_ref, *, n_cols, blk_rows):
    pid = pl.program_id(0)
    shape = (blk_rows, n_cols)
    row = jax.lax.broadcasted_iota(jnp.int32, shape, 0) + pid * blk_rows
    col = jax.lax.broadcasted_iota(jnp.int32, shape, 1)
    flat = (row * n_cols + col).astype(jnp.uint32)

    bits = _threefry_bits(flat)
    fbits = (bits >> np.uint32(9)) | np.uint32(0x3F800000)
    floats = jax.lax.bitcast_convert_type(fbits, jnp.float32) - np.float32(1.0)
    u = jnp.maximum(_TINY, floats * _SCALE + _TINY)
    gumbel = -jnp.log(-jnp.log(u))

    t = gumbel + jnp.log(x_ref[...] + np.float32(1e-10))
    m = jnp.max(t, axis=1, keepdims=True)
    idx = jnp.min(jnp.where(t == m, col, jnp.int32(np.iinfo(np.int32).max)),
                  axis=1, keepdims=True)
    o_ref[...] = (col == idx).astype(jnp.float32)


@jax.jit
def kernel(x):
    n_rows, n_cols = x.shape
    blk_rows = 8
    grid = (n_rows // blk_rows,)
    return pl.pallas_call(
        functools.partial(_sampler_kernel, n_cols=n_cols, blk_rows=blk_rows),
        grid=grid,
        in_specs=[pl.BlockSpec((blk_rows, n_cols), lambda i: (i, 0))],
        out_specs=pl.BlockSpec((blk_rows, n_cols), lambda i: (i, 0)),
        out_shape=jax.ShapeDtypeStruct((n_rows, n_cols), jnp.float32),
    )(x)
